# baseline (speedup 1.0000x reference)
import functools

import jax
import jax.numpy as jnp
from jax import lax
from jax.experimental import pallas as pl
from jax.experimental.pallas import tpu as pltpu
from jax.experimental.pallas import tpu_sc as plsc

_M = 8192
_K = 2048
_E = 16
_BM = 1024

_NW = 32
_CHUNK = _M // _NW


def _router_body(x_ref, w_ref, gate_ref):
    x = x_ref[...]
    w = w_ref[...]
    lt = jax.lax.dot_general(
        w, x, (((1,), (1,)), ((), ())), preferred_element_type=jnp.float32
    )
    m = jnp.max(lt, axis=0, keepdims=True)
    e = jnp.exp(lt - m)
    s = jnp.sum(e, axis=0, keepdims=True)
    gate_ref[...] = e / s


@functools.partial(
    pl.kernel,
    out_type=[
        jax.ShapeDtypeStruct((_M,), jnp.float32),
        jax.ShapeDtypeStruct((_M,), jnp.float32),
        jax.ShapeDtypeStruct((_M,), jnp.int32),
        jax.ShapeDtypeStruct((_M,), jnp.int32),
    ],
    mesh=plsc.VectorSubcoreMesh(
        core_axis_name="c", subcore_axis_name="s", num_cores=2, num_subcores=16
    ),
    scratch_types=[
        pltpu.VMEM((_E, _CHUNK), jnp.float32),
        pltpu.VMEM((_CHUNK,), jnp.float32),
        pltpu.VMEM((_CHUNK,), jnp.float32),
        pltpu.VMEM((_CHUNK,), jnp.int32),
        pltpu.VMEM((_CHUNK,), jnp.int32),
        pltpu.SemaphoreType.DMA,
    ],
)
def _sc_top2(gt_hbm, v1_hbm, v2_hbm, i1_hbm, i2_hbm,
             gbuf, v1b, v2b, i1b, i2b, sem):
    wid = lax.axis_index("s") * 2 + lax.axis_index("c")
    base = wid * _CHUNK
    pltpu.async_copy(gt_hbm.at[:, pl.ds(base, _CHUNK)], gbuf, sem).wait()
    for g in range(_CHUNK // 16):
        sl = pl.ds(g * 16, 16)
        ge = [gbuf[e, sl] for e in range(_E)]
        v1 = ge[0]
        i1 = jnp.zeros((16,), jnp.float32)
        for e in range(1, _E):
            better = ge[e] > v1
            v1 = jnp.where(better, ge[e], v1)
            i1 = jnp.where(better, float(e), i1)
        v2 = jnp.full((16,), -1.0, jnp.float32)
        i2 = jnp.zeros((16,), jnp.float32)
        for e in range(_E):
            ok = jnp.logical_and(i1 != float(e), ge[e] > v2)
            v2 = jnp.where(ok, ge[e], v2)
            i2 = jnp.where(ok, float(e), i2)
        v1b[sl] = v1
        v2b[sl] = v2
        i1b[sl] = i1.astype(jnp.int32)
        i2b[sl] = i2.astype(jnp.int32)
    pltpu.sync_copy(v1b, v1_hbm.at[pl.ds(base, _CHUNK)])
    pltpu.sync_copy(v2b, v2_hbm.at[pl.ds(base, _CHUNK)])
    pltpu.sync_copy(i1b, i1_hbm.at[pl.ds(base, _CHUNK)])
    pltpu.sync_copy(i2b, i2_hbm.at[pl.ds(base, _CHUNK)])


@jax.jit
def kernel(x, W):
    grid = (_M // _BM,)
    gate_t = pl.pallas_call(
        _router_body,
        grid=grid,
        in_specs=[
            pl.BlockSpec((_BM, _K), lambda i: (i, 0)),
            pl.BlockSpec((_E, _K), lambda i: (0, 0)),
        ],
        out_specs=pl.BlockSpec((_E, _BM), lambda i: (0, i)),
        out_shape=jax.ShapeDtypeStruct((_E, _M), jnp.float32),
        compiler_params=pltpu.CompilerParams(
            dimension_semantics=("parallel",),
        ),
    )(x, W)
    v1, v2, i1, i2 = _sc_top2(gate_t)
    val = jnp.stack([v1, v2], axis=1)
    idx = jnp.stack([i1, i2], axis=1)
    return (val, idx, gate_t.T)


# speedup vs baseline: 1.9168x; 1.9168x over previous
import jax
import jax.numpy as jnp
from jax.experimental import pallas as pl
from jax.experimental.pallas import tpu as pltpu

_M = 8192
_K = 2048
_E = 16
_BM = 1024


def _router_body(x_ref, w_ref, gate_ref, val_ref, idx_ref):
    x = x_ref[...]
    w = w_ref[...]
    lt = jax.lax.dot_general(
        w, x, (((1,), (1,)), ((), ())), preferred_element_type=jnp.float32
    )
    m = jnp.max(lt, axis=0, keepdims=True)
    e = jnp.exp(lt - m)
    s = jnp.sum(e, axis=0, keepdims=True)
    gt = e / s
    gate_ref[...] = gt

    lanef = jax.lax.broadcasted_iota(jnp.int32, gt.shape, 0).astype(jnp.float32)
    v1 = jnp.max(gt, axis=0, keepdims=True)
    i1 = jnp.min(jnp.where(gt == v1, lanef, 16.0), axis=0, keepdims=True)
    g2 = jnp.where(lanef == i1, -1.0, gt)
    v2 = jnp.max(g2, axis=0, keepdims=True)
    i2 = jnp.min(jnp.where(g2 == v2, lanef, 16.0), axis=0, keepdims=True)

    val_ref[...] = jnp.concatenate([v1, v2], axis=0)
    idx_ref[...] = jnp.concatenate([i1, i2], axis=0).astype(jnp.int32)


@jax.jit
def kernel(x, W):
    grid = (_M // _BM,)
    gate_t, val_t, idx_t = pl.pallas_call(
        _router_body,
        grid=grid,
        in_specs=[
            pl.BlockSpec((_BM, _K), lambda i: (i, 0)),
            pl.BlockSpec((_E, _K), lambda i: (0, 0)),
        ],
        out_specs=[
            pl.BlockSpec((_E, _BM), lambda i: (0, i)),
            pl.BlockSpec((2, _BM), lambda i: (0, i)),
            pl.BlockSpec((2, _BM), lambda i: (0, i)),
        ],
        out_shape=[
            jax.ShapeDtypeStruct((_E, _M), jnp.float32),
            jax.ShapeDtypeStruct((2, _M), jnp.float32),
            jax.ShapeDtypeStruct((2, _M), jnp.int32),
        ],
        compiler_params=pltpu.CompilerParams(
            dimension_semantics=("parallel",),
        ),
    )(x, W)
    return (val_t.T, idx_t.T, gate_t.T)
